# Initial kernel scaffold; baseline (speedup 1.0000x reference)
import jax
import jax.numpy as jnp
from jax.experimental import pallas as pl


def kernel(xyz_target, xyz_source, feats_target, feats_source, W1, W2):
    raise NotImplementedError("write your pallas kernel here")


# speedup vs baseline: 19.4830x; 19.4830x over previous
import functools

import jax
import jax.numpy as jnp
from jax.experimental import pallas as pl

B, NT, NS = 16, 4096, 1024
CT, CS = 256, 512
C1, C2 = 256, 256
TB = 512


def _g_kernel(fs_ref, w1a_ref, g_ref):
    g_ref[0] = jnp.dot(fs_ref[0], w1a_ref[...],
                       preferred_element_type=jnp.float32)


def _fp_kernel(xt_ref, xst_ref, ft_ref, g_ref, w1b_ref, w2_ref, out_ref):
    xt = xt_ref[0]
    xst = xst_ref[0]
    diff0 = xt[:, 0:1] - xst[0:1, :]
    d2 = diff0 * diff0
    diff1 = xt[:, 1:2] - xst[1:2, :]
    d2 = d2 + diff1 * diff1
    diff2 = xt[:, 2:3] - xst[2:3, :]
    d2 = d2 + diff2 * diff2

    lane = jax.lax.broadcasted_iota(jnp.int32, (TB, NS), 1)
    d2w = d2
    ohs = []
    recips = []
    for _ in range(3):
        m = jnp.min(d2w, axis=1, keepdims=True)
        am = jnp.argmin(d2w, axis=1)
        oh = lane == am[:, None]
        d2w = jnp.where(oh, jnp.float32(jnp.inf), d2w)
        d = jnp.maximum(jnp.sqrt(m), 1e-10)
        recips.append(1.0 / d)
        ohs.append(oh)
    norm = recips[0] + recips[1] + recips[2]
    w0 = recips[0] / norm
    w1 = recips[1] / norm
    w2 = recips[2] / norm
    ws = (w0 + w1 + w2) + 1e-6
    wsel = (jnp.where(ohs[0], w0 / ws, 0.0)
            + jnp.where(ohs[1], w1 / ws, 0.0)
            + jnp.where(ohs[2], w2 / ws, 0.0))

    interp = jnp.dot(wsel, g_ref[0], preferred_element_type=jnp.float32)
    h = interp + jnp.dot(ft_ref[0], w1b_ref[...],
                         preferred_element_type=jnp.float32)
    h = jnp.maximum(h, 0.0)
    out = jnp.dot(h, w2_ref[...], preferred_element_type=jnp.float32)
    out_ref[0] = jnp.maximum(out, 0.0)


@jax.jit
def kernel(xyz_target, xyz_source, feats_target, feats_source, W1, W2):
    W1a = W1[:CS]
    W1b = W1[CS:]
    xst = jnp.swapaxes(xyz_source, 1, 2)

    G = pl.pallas_call(
        _g_kernel,
        grid=(B,),
        in_specs=[
            pl.BlockSpec((1, NS, CS), lambda b: (b, 0, 0)),
            pl.BlockSpec((CS, C1), lambda b: (0, 0)),
        ],
        out_specs=pl.BlockSpec((1, NS, C1), lambda b: (b, 0, 0)),
        out_shape=jax.ShapeDtypeStruct((B, NS, C1), jnp.float32),
    )(feats_source, W1a)

    out = pl.pallas_call(
        _fp_kernel,
        grid=(B, NT // TB),
        in_specs=[
            pl.BlockSpec((1, TB, 3), lambda b, t: (b, t, 0)),
            pl.BlockSpec((1, 3, NS), lambda b, t: (b, 0, 0)),
            pl.BlockSpec((1, TB, CT), lambda b, t: (b, t, 0)),
            pl.BlockSpec((1, NS, C1), lambda b, t: (b, 0, 0)),
            pl.BlockSpec((CT, C1), lambda b, t: (0, 0)),
            pl.BlockSpec((C1, C2), lambda b, t: (0, 0)),
        ],
        out_specs=pl.BlockSpec((1, TB, C2), lambda b, t: (b, t, 0)),
        out_shape=jax.ShapeDtypeStruct((B, NT, C2), jnp.float32),
    )(xyz_target, xst, feats_target, G, W1b, W2)
    return out


# speedup vs baseline: 40.3193x; 2.0695x over previous
import functools

import jax
import jax.numpy as jnp
from jax.experimental import pallas as pl

B, NT, NS = 16, 4096, 1024
CT, CS = 256, 512
C1, C2 = 256, 256
TB = 512


def _g_kernel(fs_ref, w1a_ref, g_ref):
    g_ref[0] = jnp.dot(fs_ref[0], w1a_ref[...],
                       preferred_element_type=jnp.float32)


def _fp_kernel(xt_ref, xst_ref, ft_ref, g_ref, w1b_ref, w2_ref, out_ref):
    xt = xt_ref[0]
    xst = xst_ref[0]
    diff0 = xt[:, 0:1] - xst[0:1, :]
    d2 = diff0 * diff0
    diff1 = xt[:, 1:2] - xst[1:2, :]
    d2 = d2 + diff1 * diff1
    diff2 = xt[:, 2:3] - xst[2:3, :]
    d2 = d2 + diff2 * diff2

    d2w = d2
    ohs = []
    recips = []
    for _ in range(3):
        m = jnp.min(d2w, axis=1, keepdims=True)
        oh = d2w == m
        d2w = jnp.where(oh, jnp.float32(jnp.inf), d2w)
        d = jnp.maximum(jnp.sqrt(m), 1e-10)
        recips.append(1.0 / d)
        ohs.append(oh)
    norm = recips[0] + recips[1] + recips[2]
    w0 = recips[0] / norm
    w1 = recips[1] / norm
    w2 = recips[2] / norm
    ws = (w0 + w1 + w2) + 1e-6
    wsel = jnp.where(ohs[0], w0 / ws, 0.0)
    wsel = jnp.where(ohs[1], w1 / ws, wsel)
    wsel = jnp.where(ohs[2], w2 / ws, wsel)

    interp = jnp.dot(wsel, g_ref[0], preferred_element_type=jnp.float32)
    h = interp + jnp.dot(ft_ref[0], w1b_ref[...],
                         preferred_element_type=jnp.float32)
    h = jnp.maximum(h, 0.0)
    out = jnp.dot(h, w2_ref[...], preferred_element_type=jnp.float32)
    out_ref[0] = jnp.maximum(out, 0.0)


@jax.jit
def kernel(xyz_target, xyz_source, feats_target, feats_source, W1, W2):
    W1a = W1[:CS]
    W1b = W1[CS:]
    xst = jnp.swapaxes(xyz_source, 1, 2)

    G = pl.pallas_call(
        _g_kernel,
        grid=(B,),
        in_specs=[
            pl.BlockSpec((1, NS, CS), lambda b: (b, 0, 0)),
            pl.BlockSpec((CS, C1), lambda b: (0, 0)),
        ],
        out_specs=pl.BlockSpec((1, NS, C1), lambda b: (b, 0, 0)),
        out_shape=jax.ShapeDtypeStruct((B, NS, C1), jnp.float32),
    )(feats_source, W1a)

    out = pl.pallas_call(
        _fp_kernel,
        grid=(B, NT // TB),
        in_specs=[
            pl.BlockSpec((1, TB, 3), lambda b, t: (b, t, 0)),
            pl.BlockSpec((1, 3, NS), lambda b, t: (b, 0, 0)),
            pl.BlockSpec((1, TB, CT), lambda b, t: (b, t, 0)),
            pl.BlockSpec((1, NS, C1), lambda b, t: (b, 0, 0)),
            pl.BlockSpec((CT, C1), lambda b, t: (0, 0)),
            pl.BlockSpec((C1, C2), lambda b, t: (0, 0)),
        ],
        out_specs=pl.BlockSpec((1, TB, C2), lambda b, t: (b, t, 0)),
        out_shape=jax.ShapeDtypeStruct((B, NT, C2), jnp.float32),
    )(xyz_target, xst, feats_target, G, W1b, W2)
    return out


# speedup vs baseline: 43.1890x; 1.0712x over previous
import functools

import jax
import jax.numpy as jnp
from jax.experimental import pallas as pl

B, NT, NS = 16, 4096, 1024
CT, CS = 256, 512
C1, C2 = 256, 256
TB = 512


def _g_kernel(fs_ref, w1a_ref, g_ref):
    g_ref[0] = jnp.dot(fs_ref[0], w1a_ref[...],
                       preferred_element_type=jnp.float32)


def _fp_kernel(xt_ref, xst_ref, ft_ref, g_ref, w1b_ref, w2_ref, out_ref):
    xt = xt_ref[0]
    xst = xst_ref[0]
    diff0 = xt[:, 0:1] - xst[0:1, :]
    d2 = diff0 * diff0
    diff1 = xt[:, 1:2] - xst[1:2, :]
    d2 = d2 + diff1 * diff1
    diff2 = xt[:, 2:3] - xst[2:3, :]
    d2 = d2 + diff2 * diff2

    d2w = d2
    ohs = []
    recips = []
    for _ in range(3):
        m = jnp.min(d2w, axis=1, keepdims=True)
        oh = d2w == m
        d2w = jnp.where(oh, jnp.float32(jnp.inf), d2w)
        recips.append(jax.lax.rsqrt(jnp.maximum(m, 1e-20)))
        ohs.append(oh)
    r0, r1, r2 = recips
    norm = r0 + r1 + r2
    rn = 1.0 / norm
    ws = (r0 + r1 + r2) * rn + 1e-6
    c = rn / ws
    wsel = jnp.where(ohs[0], r0 * c, 0.0)
    wsel = jnp.where(ohs[1], r1 * c, wsel)
    wsel = jnp.where(ohs[2], r2 * c, wsel)

    interp = jnp.dot(wsel, g_ref[0], preferred_element_type=jnp.float32)
    h = interp + jnp.dot(ft_ref[0], w1b_ref[...],
                         preferred_element_type=jnp.float32)
    h = jnp.maximum(h, 0.0)
    out = jnp.dot(h, w2_ref[...], preferred_element_type=jnp.float32)
    out_ref[0] = jnp.maximum(out, 0.0)


@jax.jit
def kernel(xyz_target, xyz_source, feats_target, feats_source, W1, W2):
    W1a = W1[:CS]
    W1b = W1[CS:]
    xst = jnp.swapaxes(xyz_source, 1, 2)

    G = pl.pallas_call(
        _g_kernel,
        grid=(B,),
        in_specs=[
            pl.BlockSpec((1, NS, CS), lambda b: (b, 0, 0)),
            pl.BlockSpec((CS, C1), lambda b: (0, 0)),
        ],
        out_specs=pl.BlockSpec((1, NS, C1), lambda b: (b, 0, 0)),
        out_shape=jax.ShapeDtypeStruct((B, NS, C1), jnp.float32),
    )(feats_source, W1a)

    out = pl.pallas_call(
        _fp_kernel,
        grid=(B, NT // TB),
        in_specs=[
            pl.BlockSpec((1, TB, 3), lambda b, t: (b, t, 0)),
            pl.BlockSpec((1, 3, NS), lambda b, t: (b, 0, 0)),
            pl.BlockSpec((1, TB, CT), lambda b, t: (b, t, 0)),
            pl.BlockSpec((1, NS, C1), lambda b, t: (b, 0, 0)),
            pl.BlockSpec((CT, C1), lambda b, t: (0, 0)),
            pl.BlockSpec((C1, C2), lambda b, t: (0, 0)),
        ],
        out_specs=pl.BlockSpec((1, TB, C2), lambda b, t: (b, t, 0)),
        out_shape=jax.ShapeDtypeStruct((B, NT, C2), jnp.float32),
    )(xyz_target, xst, feats_target, G, W1b, W2)
    return out


# speedup vs baseline: 48.3485x; 1.1195x over previous
import functools

import jax
import jax.numpy as jnp
from jax.experimental import pallas as pl

B, NT, NS = 16, 4096, 1024
CT, CS = 256, 512
C1, C2 = 256, 256
TB = 1024


def _g_kernel(fs_ref, w1a_ref, g_ref):
    g_ref[0] = jnp.dot(fs_ref[0], w1a_ref[...],
                       preferred_element_type=jnp.float32)


def _fp_kernel(xt_ref, xst_ref, ft_ref, g_ref, w1b_ref, w2_ref, out_ref):
    xt = xt_ref[0]
    xst = xst_ref[0]
    diff0 = xt[:, 0:1] - xst[0:1, :]
    d2 = diff0 * diff0
    diff1 = xt[:, 1:2] - xst[1:2, :]
    d2 = d2 + diff1 * diff1
    diff2 = xt[:, 2:3] - xst[2:3, :]
    d2 = d2 + diff2 * diff2

    d2w = d2
    ohs = []
    recips = []
    for k in range(3):
        m = jnp.min(d2w, axis=1, keepdims=True)
        oh = d2w == m
        if k < 2:
            d2w = jnp.where(oh, jnp.float32(jnp.inf), d2w)
        recips.append(jax.lax.rsqrt(jnp.maximum(m, 1e-20)))
        ohs.append(oh)
    r0, r1, r2 = recips
    norm = r0 + r1 + r2
    rn = 1.0 / norm
    ws = (r0 + r1 + r2) * rn + 1e-6
    c = rn / ws
    wsel = jnp.where(ohs[0], r0 * c, 0.0)
    wsel = jnp.where(ohs[1], r1 * c, wsel)
    wsel = jnp.where(ohs[2], r2 * c, wsel)

    interp = jnp.dot(wsel, g_ref[0], preferred_element_type=jnp.float32)
    h = interp + jnp.dot(ft_ref[0], w1b_ref[...],
                         preferred_element_type=jnp.float32)
    h = jnp.maximum(h, 0.0)
    out = jnp.dot(h, w2_ref[...], preferred_element_type=jnp.float32)
    out_ref[0] = jnp.maximum(out, 0.0)


@jax.jit
def kernel(xyz_target, xyz_source, feats_target, feats_source, W1, W2):
    W1a = W1[:CS]
    W1b = W1[CS:]
    xst = jnp.swapaxes(xyz_source, 1, 2)

    G = pl.pallas_call(
        _g_kernel,
        grid=(B,),
        in_specs=[
            pl.BlockSpec((1, NS, CS), lambda b: (b, 0, 0)),
            pl.BlockSpec((CS, C1), lambda b: (0, 0)),
        ],
        out_specs=pl.BlockSpec((1, NS, C1), lambda b: (b, 0, 0)),
        out_shape=jax.ShapeDtypeStruct((B, NS, C1), jnp.float32),
    )(feats_source, W1a)

    out = pl.pallas_call(
        _fp_kernel,
        grid=(B, NT // TB),
        in_specs=[
            pl.BlockSpec((1, TB, 3), lambda b, t: (b, t, 0)),
            pl.BlockSpec((1, 3, NS), lambda b, t: (b, 0, 0)),
            pl.BlockSpec((1, TB, CT), lambda b, t: (b, t, 0)),
            pl.BlockSpec((1, NS, C1), lambda b, t: (b, 0, 0)),
            pl.BlockSpec((CT, C1), lambda b, t: (0, 0)),
            pl.BlockSpec((C1, C2), lambda b, t: (0, 0)),
        ],
        out_specs=pl.BlockSpec((1, TB, C2), lambda b, t: (b, t, 0)),
        out_shape=jax.ShapeDtypeStruct((B, NT, C2), jnp.float32),
    )(xyz_target, xst, feats_target, G, W1b, W2)
    return out
